# Initial kernel scaffold; baseline (speedup 1.0000x reference)
import jax
import jax.numpy as jnp
from jax.experimental import pallas as pl


def kernel(x, edge_attr, edge_index, batch, params):
    raise NotImplementedError("write your pallas kernel here")


# speedup vs baseline: 1.2671x; 1.2671x over previous
import functools
import jax
import jax.numpy as jnp
from jax.experimental import pallas as pl
from jax.experimental.pallas import tpu as pltpu

N = 10000
E = 160000
G = 64
D = 300
NB = 1000
EB = 2000
NBLK = N // NB
EBLK = E // EB
EPS = 1e-5


def _enc_body(idx_ref, tab_ref, o_ref, *, nfield, card):
    ids = idx_ref[0]
    b = ids.shape[0]
    oh = []
    for i in range(nfield):
        col = ids[:, i][:, None]
        io = jax.lax.broadcasted_iota(jnp.int32, (b, card), 1)
        oh.append((col == io).astype(jnp.float32))
    onehot = jnp.concatenate(oh, axis=1)
    o_ref[...] = jnp.dot(onehot, tab_ref[...],
                         preferred_element_type=jnp.float32,
                         precision=jax.lax.Precision.HIGHEST)


def _encode(ids, table, nfield, card, blk):
    n = ids.shape[0]
    nblk = n // blk
    ids3 = ids.reshape(nblk, blk, nfield).astype(jnp.int32)
    return pl.pallas_call(
        functools.partial(_enc_body, nfield=nfield, card=card),
        grid=(nblk,),
        in_specs=[
            pl.BlockSpec((1, blk, nfield), lambda i: (i, 0, 0)),
            pl.BlockSpec((nfield * card, D), lambda i: (0, 0)),
        ],
        out_specs=pl.BlockSpec((blk, D), lambda i: (i, 0)),
        out_shape=jax.ShapeDtypeStruct((n, D), jnp.float32),
    )(ids3, table)


def _addvn_body(h_ref, b_ref, vn_ref, o_ref):
    bvec = b_ref[0, 0, :]
    io = jax.lax.broadcasted_iota(jnp.int32, (NB, G), 1)
    onehot = (bvec[:, None] == io).astype(jnp.float32)
    o_ref[...] = h_ref[...] + jnp.dot(onehot, vn_ref[...],
                                      preferred_element_type=jnp.float32,
                                      precision=jax.lax.Precision.HIGHEST)


def _add_vn(h, batch3, vn):
    return pl.pallas_call(
        _addvn_body,
        grid=(NBLK,),
        in_specs=[
            pl.BlockSpec((NB, D), lambda i: (i, 0)),
            pl.BlockSpec((1, 1, NB), lambda i: (i, 0, 0)),
            pl.BlockSpec((G, D), lambda i: (0, 0)),
        ],
        out_specs=pl.BlockSpec((NB, D), lambda i: (i, 0)),
        out_shape=jax.ShapeDtypeStruct((N, D), jnp.float32),
    )(h, batch3, vn)


def _scatter_body(src_ref, dst_ref, hv_ref, ea_ref, o_ref):
    @pl.when(pl.program_id(0) == 0)
    def _():
        o_ref[...] = jnp.zeros_like(o_ref)

    def body(e, _):
        s = src_ref[0, 0, e]
        d = dst_ref[0, 0, e]
        row = hv_ref[pl.ds(s, 1), :] + ea_ref[pl.ds(e, 1), :]
        msg = jnp.maximum(row, 0.0)
        o_ref[pl.ds(d, 1), :] = o_ref[pl.ds(d, 1), :] + msg
        return 0

    jax.lax.fori_loop(0, EB, body, 0)


def _aggregate(hv, ea, src2, dst2):
    return pl.pallas_call(
        _scatter_body,
        grid=(EBLK,),
        in_specs=[
            pl.BlockSpec((1, 1, EB), lambda i: (i, 0, 0),
                         memory_space=pltpu.SMEM),
            pl.BlockSpec((1, 1, EB), lambda i: (i, 0, 0),
                         memory_space=pltpu.SMEM),
            pl.BlockSpec((N, D), lambda i: (0, 0)),
            pl.BlockSpec((EB, D), lambda i: (i, 0)),
        ],
        out_specs=pl.BlockSpec((N, D), lambda i: (0, 0)),
        out_shape=jax.ShapeDtypeStruct((N, D), jnp.float32),
    )(src2, dst2, hv, ea)


def _mlp1_body(hv_ref, ag_ref, w_ref, b_ref, z_ref, st_ref):
    z = jnp.dot(hv_ref[...] + ag_ref[...], w_ref[...],
                preferred_element_type=jnp.float32) + b_ref[...]
    z_ref[...] = z

    @pl.when(pl.program_id(0) == 0)
    def _():
        st_ref[...] = jnp.zeros_like(st_ref)

    st_ref[...] += jnp.stack([jnp.sum(z, 0), jnp.sum(z * z, 0)], 0)


def _mlp1(hv, aggr, w1, b1):
    return pl.pallas_call(
        _mlp1_body,
        grid=(NBLK,),
        in_specs=[
            pl.BlockSpec((NB, D), lambda i: (i, 0)),
            pl.BlockSpec((NB, D), lambda i: (i, 0)),
            pl.BlockSpec((D, D), lambda i: (0, 0)),
            pl.BlockSpec((1, D), lambda i: (0, 0)),
        ],
        out_specs=[
            pl.BlockSpec((NB, D), lambda i: (i, 0)),
            pl.BlockSpec((2, D), lambda i: (0, 0)),
        ],
        out_shape=[
            jax.ShapeDtypeStruct((N, D), jnp.float32),
            jax.ShapeDtypeStruct((2, D), jnp.float32),
        ],
    )(hv, aggr, w1, b1)


def _cstats_body(z_ref, st_ref, o_ref):
    mu = st_ref[0:1, :] / N
    d = z_ref[...] - mu

    @pl.when(pl.program_id(0) == 0)
    def _():
        o_ref[...] = jnp.zeros_like(o_ref)

    o_ref[...] += jnp.stack([jnp.sum(d * d, 0), jnp.zeros((D,), jnp.float32)], 0)


def _cstats(z, st):
    return pl.pallas_call(
        _cstats_body,
        grid=(NBLK,),
        in_specs=[
            pl.BlockSpec((NB, D), lambda i: (i, 0)),
            pl.BlockSpec((2, D), lambda i: (0, 0)),
        ],
        out_specs=pl.BlockSpec((2, D), lambda i: (0, 0)),
        out_shape=jax.ShapeDtypeStruct((2, D), jnp.float32),
    )(z, st)


def _mlp2_body(z1_ref, st_ref, cs_ref, g_ref, bt_ref, w_ref, b_ref, z_ref, st2_ref):
    mu = st_ref[0:1, :] / N
    var = cs_ref[0:1, :] / N
    inv = jax.lax.rsqrt(var + EPS)
    y = (z1_ref[...] - mu) * inv * g_ref[...] + bt_ref[...]
    y = jnp.maximum(y, 0.0)
    z = jnp.dot(y, w_ref[...], preferred_element_type=jnp.float32) + b_ref[...]
    z_ref[...] = z

    @pl.when(pl.program_id(0) == 0)
    def _():
        st2_ref[...] = jnp.zeros_like(st2_ref)

    st2_ref[...] += jnp.stack([jnp.sum(z, 0), jnp.sum(z * z, 0)], 0)


def _mlp2(z1, st1, cs1, g1, bt1, w2, b2):
    return pl.pallas_call(
        _mlp2_body,
        grid=(NBLK,),
        in_specs=[
            pl.BlockSpec((NB, D), lambda i: (i, 0)),
            pl.BlockSpec((2, D), lambda i: (0, 0)),
            pl.BlockSpec((2, D), lambda i: (0, 0)),
            pl.BlockSpec((1, D), lambda i: (0, 0)),
            pl.BlockSpec((1, D), lambda i: (0, 0)),
            pl.BlockSpec((D, D), lambda i: (0, 0)),
            pl.BlockSpec((1, D), lambda i: (0, 0)),
        ],
        out_specs=[
            pl.BlockSpec((NB, D), lambda i: (i, 0)),
            pl.BlockSpec((2, D), lambda i: (0, 0)),
        ],
        out_shape=[
            jax.ShapeDtypeStruct((N, D), jnp.float32),
            jax.ShapeDtypeStruct((2, D), jnp.float32),
        ],
    )(z1, st1, cs1, g1, bt1, w2, b2)


def _res_body(z2_ref, st_ref, cs_ref, g_ref, bt_ref, hv_ref, b_ref, h_ref, p_ref, c_ref):
    mu = st_ref[0:1, :] / N
    var = cs_ref[0:1, :] / N
    inv = jax.lax.rsqrt(var + EPS)
    y = (z2_ref[...] - mu) * inv * g_ref[...] + bt_ref[...]
    h = jnp.maximum(y, 0.0) + hv_ref[...]
    h_ref[...] = h

    bvec = b_ref[0, 0, :]
    io = jax.lax.broadcasted_iota(jnp.int32, (NB, G), 1)
    onehot = (bvec[:, None] == io).astype(jnp.float32)

    @pl.when(pl.program_id(0) == 0)
    def _():
        p_ref[...] = jnp.zeros_like(p_ref)
        c_ref[...] = jnp.zeros_like(c_ref)

    p_ref[...] += jnp.dot(onehot.T, h, preferred_element_type=jnp.float32,
                          precision=jax.lax.Precision.HIGHEST)
    c_ref[...] += jnp.sum(onehot, axis=0)[:, None]


def _res_pool(z2, st2, cs2, g, b, hv, batch3):
    return pl.pallas_call(
        _res_body,
        grid=(NBLK,),
        in_specs=[
            pl.BlockSpec((NB, D), lambda i: (i, 0)),
            pl.BlockSpec((2, D), lambda i: (0, 0)),
            pl.BlockSpec((2, D), lambda i: (0, 0)),
            pl.BlockSpec((1, D), lambda i: (0, 0)),
            pl.BlockSpec((1, D), lambda i: (0, 0)),
            pl.BlockSpec((NB, D), lambda i: (i, 0)),
            pl.BlockSpec((1, 1, NB), lambda i: (i, 0, 0)),
        ],
        out_specs=[
            pl.BlockSpec((NB, D), lambda i: (i, 0)),
            pl.BlockSpec((G, D), lambda i: (0, 0)),
            pl.BlockSpec((G, 1), lambda i: (0, 0)),
        ],
        out_shape=[
            jax.ShapeDtypeStruct((N, D), jnp.float32),
            jax.ShapeDtypeStruct((G, D), jnp.float32),
            jax.ShapeDtypeStruct((G, 1), jnp.float32),
        ],
    )(z2, st2, cs2, g, b, hv, batch3)


def _vnmlp_body(vn_ref, p_ref, c_ref, w1_ref, b1_ref, g1_ref, t1_ref,
                w2_ref, b2_ref, g2_ref, t2_ref, o_ref):
    cnt = jnp.maximum(c_ref[...], 1.0)
    v = vn_ref[...] + p_ref[...] / cnt
    v = jnp.dot(v, w1_ref[...], preferred_element_type=jnp.float32) + b1_ref[...]
    mu = jnp.mean(v, 0, keepdims=True)
    var = jnp.mean(v * v, 0, keepdims=True) - mu * mu
    v = (v - mu) * jax.lax.rsqrt(var + EPS) * g1_ref[...] + t1_ref[...]
    v = jnp.maximum(v, 0.0)
    v = jnp.dot(v, w2_ref[...], preferred_element_type=jnp.float32) + b2_ref[...]
    mu = jnp.mean(v, 0, keepdims=True)
    var = jnp.mean(v * v, 0, keepdims=True) - mu * mu
    v = (v - mu) * jax.lax.rsqrt(var + EPS) * g2_ref[...] + t2_ref[...]
    o_ref[...] = jnp.maximum(v, 0.0)


def _vn_mlp(vn, pool, cnt, m):
    full = lambda *_: tuple(0 for _ in range(2))
    specs = [pl.BlockSpec(a.shape, full) for a in
             (vn, pool, cnt, m['W1'], m['b1'], m['g1'], m['bt1'],
              m['W2'], m['b2'], m['g2'], m['bt2'])]
    return pl.pallas_call(
        _vnmlp_body,
        in_specs=specs,
        out_specs=pl.BlockSpec((G, D), lambda *_: (0, 0)),
        out_shape=jax.ShapeDtypeStruct((G, D), jnp.float32),
    )(vn, pool, cnt, m['W1'], m['b1'], m['g1'], m['bt1'],
      m['W2'], m['b2'], m['g2'], m['bt2'])


def _read_body(p_ref, c_ref, w_ref, b_ref, o_ref):
    g = p_ref[...] / jnp.maximum(c_ref[...], 1.0)
    o_ref[...] = jnp.dot(g, w_ref[...],
                         preferred_element_type=jnp.float32) + b_ref[...]


def _readout(pool, cnt, w, b):
    full = lambda *_: (0, 0)
    return pl.pallas_call(
        _read_body,
        in_specs=[pl.BlockSpec(pool.shape, full),
                  pl.BlockSpec(cnt.shape, full),
                  pl.BlockSpec(w.shape, full),
                  pl.BlockSpec(b.shape, full)],
        out_specs=pl.BlockSpec((G, 1), full),
        out_shape=jax.ShapeDtypeStruct((G, 1), jnp.float32),
    )(pool, cnt, w, b)


def kernel(x, edge_attr, edge_index, batch, params):
    p = params
    atab = jnp.concatenate([t[:12] for t in p['atom_emb']], axis=0)
    btab = jnp.concatenate([t[:12] for t in p['bond_emb']], axis=0)

    h = _encode(x, atab, 9, 12, NB)
    ea = _encode(edge_attr, btab, 3, 12, EB)

    src2 = edge_index[0].reshape(EBLK, 1, EB).astype(jnp.int32)
    dst2 = edge_index[1].reshape(EBLK, 1, EB).astype(jnp.int32)
    batch3 = batch.reshape(NBLK, 1, NB).astype(jnp.int32)

    vn = jnp.broadcast_to(p['vn_emb'][0], (G, D))
    row = lambda v: v.reshape(1, D)

    for l in range(3):
        hv = _add_vn(h, batch3, vn)
        aggr = _aggregate(hv, ea, src2, dst2)
        c = p['convs'][l]
        z1, st1 = _mlp1(hv, aggr, c['W1'], row(c['b1']))
        cs1 = _cstats(z1, st1)
        z2, st2 = _mlp2(z1, st1, cs1, row(c['g1']), row(c['bt1']),
                        c['W2'], row(c['b2']))
        cs2 = _cstats(z2, st2)
        h, pool, cnt = _res_pool(z2, st2, cs2, row(p['bns'][l]['g']),
                                 row(p['bns'][l]['b']), hv, batch3)
        if l < 2:
            m = p['vn_mlps'][l]
            vn = _vn_mlp(vn, pool, cnt,
                         {k: (v.reshape(1, D) if v.ndim == 1 else v)
                          for k, v in m.items()})

    return _readout(pool, cnt, p['lin']['W'], p['lin']['b'].reshape(1, 1))
